# baseline (speedup 1.0000x reference)
import jax
import jax.numpy as jnp
from jax.experimental import pallas as pl
from jax.experimental.pallas import tpu as pltpu


def _add_kernel(x_ref, emb_ref, out_ref):
    out_ref[...] = x_ref[...] + emb_ref[...][None, :, :]


def kernel(x, emb_table):
    batch, seq, d = x.shape
    sb = 2048
    n_seq = seq // sb

    return pl.pallas_call(
        _add_kernel,
        grid=(n_seq, batch),
        in_specs=[
            pl.BlockSpec((1, sb, d), lambda s, b: (b, s, 0)),
            pl.BlockSpec((sb, d), lambda s, b: (s, 0)),
        ],
        out_specs=pl.BlockSpec((1, sb, d), lambda s, b: (b, s, 0)),
        out_shape=jax.ShapeDtypeStruct((batch, seq, d), x.dtype),
        compiler_params=pltpu.CompilerParams(
            vmem_limit_bytes=120 * 1024 * 1024,
        ),
        input_output_aliases={0: 0},
    )(x, emb_table)
